# baseline (speedup 1.0000x reference)
import functools

import jax
import jax.numpy as jnp
from jax import lax
from jax.experimental import pallas as pl
from jax.experimental.pallas import tpu as pltpu
from jax.experimental.pallas import tpu_sc as plsc

NC = 2
NS = 16
NW = NC * NS
L = 16
CHUNK = 800
NBUF = 2


@functools.lru_cache(maxsize=None)
def _make_gather(n_chunks, v, d):
    mesh = plsc.VectorSubcoreMesh(core_axis_name="c", subcore_axis_name="s")
    n_rows = n_chunks * CHUNK

    @functools.partial(
        pl.kernel,
        out_type=jax.ShapeDtypeStruct((NW * n_rows, d), jnp.float32),
        mesh=mesh,
        scratch_types=[
            pltpu.VMEM((n_rows,), jnp.int32),
            pltpu.VMEM((NBUF, CHUNK, d), jnp.float32),
            pltpu.SemaphoreType.DMA((NBUF,)),
            pltpu.SemaphoreType.DMA((NBUF,)),
        ],
        compiler_params=pltpu.CompilerParams(use_tc_tiling_on_sc=False),
    )
    def gather_kernel(idx_hbm, table_hbm, out_hbm, idx_v, rows_v, gsem, osem):
        wid = lax.axis_index("s") * NC + lax.axis_index("c")
        base = wid * n_rows
        pltpu.sync_copy(idx_hbm.at[wid], idx_v)

        def issue_chunk(c, b):
            @pl.loop(0, CHUNK // L, unroll=8)
            def _(k):
                iv = idx_v[pl.ds(c * CHUNK + k * L, L)]
                pltpu.async_copy(
                    table_hbm.at[iv],
                    rows_v.at[b].at[pl.ds(k * L, L)],
                    gsem.at[b],
                )

        def drain_chunk(b):
            pltpu.make_async_copy(
                table_hbm.at[pl.ds(0, CHUNK)], rows_v.at[b], gsem.at[b]
            ).wait()

        outs = [None] * n_chunks
        issue_chunk(0, 0)
        for c in range(n_chunks):
            b = c % NBUF
            drain_chunk(b)
            outs[c] = pltpu.async_copy(
                rows_v.at[b], out_hbm.at[pl.ds(base + c * CHUNK, CHUNK)],
                osem.at[b])
            if c + 1 < n_chunks:
                nb = (c + 1) % NBUF
                if c >= 1:
                    outs[c - 1].wait()
                issue_chunk(c + 1, nb)
        if n_chunks >= 2:
            outs[n_chunks - 2].wait()
        outs[n_chunks - 1].wait()

    return gather_kernel


def kernel(sequence, table):
    b, s = sequence.shape
    v, d = table.shape
    flat = sequence.reshape(-1).astype(jnp.int32)
    n = flat.shape[0]
    per_w = -(-n // (NW * CHUNK)) * CHUNK
    n_pad = NW * per_w
    if n_pad != n:
        flat = jnp.pad(flat, (0, n_pad - n))
    idx3 = flat.reshape(NW, per_w)
    out = _make_gather(per_w // CHUNK, v, d)(idx3, table)
    return out[:n].reshape(b, s, d)


# speedup vs baseline: 1.6021x; 1.6021x over previous
import functools

import jax
import jax.numpy as jnp
from jax import lax
from jax.experimental import pallas as pl
from jax.experimental.pallas import tpu as pltpu
from jax.experimental.pallas import tpu_sc as plsc

NC = 2
NS = 16
NW = NC * NS
CHUNK = 320
NBUF = 2


@functools.lru_cache(maxsize=None)
def _make_gather(n_chunks, v, d):
    mesh = plsc.VectorSubcoreMesh(core_axis_name="c", subcore_axis_name="s")
    n_rows = n_chunks * CHUNK

    @functools.partial(
        pl.kernel,
        out_type=jax.ShapeDtypeStruct((NW * n_rows, d), jnp.float32),
        mesh=mesh,
        scratch_types=[
            pltpu.VMEM((CHUNK,), jnp.int32),
            pltpu.VMEM((NBUF, CHUNK, d), jnp.float32),
            pltpu.SemaphoreType.DMA((NBUF,)),
            pltpu.SemaphoreType.DMA((NBUF,)),
        ],
    )
    def gather_kernel(idx_hbm, table_hbm, out_hbm, idx_v, rows_v,
                      gsem, osem):
        wid = lax.axis_index("s") * NC + lax.axis_index("c")
        base = wid * n_rows

        def stage_and_issue(c, b):
            pltpu.sync_copy(idx_hbm.at[pl.ds(base + c * CHUNK, CHUNK)], idx_v)

            @pl.loop(0, CHUNK // 16)
            def _(g):
                iv = idx_v[pl.ds(g * 16, 16)]
                for i in range(16):
                    pltpu.async_copy(
                        table_hbm.at[pl.ds(iv[i], 1)],
                        rows_v.at[b].at[pl.ds(g * 16 + i, 1)],
                        gsem.at[b],
                    )

        def drain(b):
            pltpu.make_async_copy(
                table_hbm.at[pl.ds(0, CHUNK)], rows_v.at[b], gsem.at[b]
            ).wait()

        outs = [None] * n_chunks
        stage_and_issue(0, 0)
        for c in range(n_chunks):
            b = c % NBUF
            drain(b)
            outs[c] = pltpu.async_copy(
                rows_v.at[b], out_hbm.at[pl.ds(base + c * CHUNK, CHUNK)],
                osem.at[b])
            if c + 1 < n_chunks:
                if c >= 1:
                    outs[c - 1].wait()
                stage_and_issue(c + 1, (c + 1) % NBUF)
        if n_chunks >= 2:
            outs[n_chunks - 2].wait()
        outs[n_chunks - 1].wait()

    return gather_kernel


def kernel(sequence, table):
    b, s = sequence.shape
    v, d = table.shape
    flat = sequence.reshape(-1).astype(jnp.int32)
    n = flat.shape[0]
    per_w = -(-n // (NW * CHUNK)) * CHUNK
    n_pad = NW * per_w
    if n_pad != n:
        flat = jnp.pad(flat, (0, n_pad - n))
    out = _make_gather(per_w // CHUNK, v, d)(flat, table)
    return out[:n].reshape(b, s, d)


# speedup vs baseline: 1.6211x; 1.0118x over previous
import functools

import jax
import jax.numpy as jnp
from jax import lax
from jax.experimental import pallas as pl
from jax.experimental.pallas import tpu as pltpu
from jax.experimental.pallas import tpu_sc as plsc

NC = 2
NS = 16
NW = NC * NS
CHUNK = 320
NBUF = 3
NSEM = 4


@functools.lru_cache(maxsize=None)
def _make_gather(n_chunks, v, d):
    mesh = plsc.VectorSubcoreMesh(core_axis_name="c", subcore_axis_name="s")
    n_rows = n_chunks * CHUNK

    @functools.partial(
        pl.kernel,
        out_type=jax.ShapeDtypeStruct((NW * n_rows, d), jnp.float32),
        mesh=mesh,
        scratch_types=[
            pltpu.VMEM((n_rows,), jnp.int32),
            pltpu.VMEM((NBUF, CHUNK, d), jnp.float32),
            pltpu.SemaphoreType.DMA((NBUF, NSEM)),
            pltpu.SemaphoreType.DMA((NBUF,)),
        ],
    )
    def gather_kernel(idx_hbm, table_hbm, out_hbm, idx_v, rows_v, gsem, osem):
        wid = lax.axis_index("s") * NC + lax.axis_index("c")
        base = wid * n_rows
        pltpu.sync_copy(idx_hbm.at[pl.ds(base, n_rows)], idx_v)

        def issue(c, b):
            @pl.loop(0, CHUNK // 16)
            def _(g):
                iv = idx_v[pl.ds(c * CHUNK + g * 16, 16)]
                for i in range(16):
                    pltpu.async_copy(
                        table_hbm.at[pl.ds(iv[i], 1)],
                        rows_v.at[b].at[pl.ds(g * 16 + i, 1)],
                        gsem.at[b].at[i % NSEM],
                    )

        def drain(b):
            for q in range(NSEM):
                pltpu.make_async_copy(
                    table_hbm.at[pl.ds(0, CHUNK // NSEM)],
                    rows_v.at[b].at[pl.ds(0, CHUNK // NSEM)],
                    gsem.at[b].at[q],
                ).wait()

        outs = [None] * n_chunks
        for c in range(min(NBUF, n_chunks)):
            issue(c, c)
        for c in range(n_chunks):
            b = c % NBUF
            drain(b)
            outs[c] = pltpu.async_copy(
                rows_v.at[b], out_hbm.at[pl.ds(base + c * CHUNK, CHUNK)],
                osem.at[b])
            if c + NBUF < n_chunks:
                outs[c].wait()
                issue(c + NBUF, b)
        for c in range(max(0, n_chunks - NBUF), n_chunks):
            outs[c].wait()

    return gather_kernel


def kernel(sequence, table):
    b, s = sequence.shape
    v, d = table.shape
    flat = sequence.reshape(-1).astype(jnp.int32)
    n = flat.shape[0]
    per_w = -(-n // (NW * CHUNK)) * CHUNK
    n_pad = NW * per_w
    if n_pad != n:
        flat = jnp.pad(flat, (0, n_pad - n))
    out = _make_gather(per_w // CHUNK, v, d)(flat, table)
    return out[:n].reshape(b, s, d)
